# baseline (speedup 1.0000x reference)
import functools

import jax
import jax.numpy as jnp
from jax import lax
from jax.experimental import pallas as pl
from jax.experimental.pallas import tpu as pltpu
from jax.experimental.pallas import tpu_sc as plsc

def _dot(a, b):
    return jnp.dot(a, b, preferred_element_type=jnp.float32,
                   precision=jax.lax.Precision.DEFAULT)


def _dot_exact(a, b):
    return jnp.dot(a, b, preferred_element_type=jnp.float32,
                   precision=jax.lax.Precision.HIGHEST)


_SC_MESH = plsc.VectorSubcoreMesh(core_axis_name="c", subcore_axis_name="s",
                                  num_cores=1, num_subcores=1)


@functools.partial(
    pl.kernel,
    mesh=_SC_MESH,
    compiler_params=pltpu.CompilerParams(needs_layout_passes=False,
                                         skip_device_barrier=True),
    out_type=jax.ShapeDtypeStruct((16,), jnp.float32),
    scratch_types=[
        pltpu.VMEM((16,), jnp.int32),
        pltpu.VMEM((16,), jnp.int32),
        pltpu.VMEM((16,), jnp.float32),
        pltpu.VMEM((16,), jnp.float32),
    ],
)
def _adjacency_sc(src_hbm, dst_hbm, a_hbm, src_v, dst_v, deg_v, a_v):
    wid = lax.axis_index("s") * 2 + lax.axis_index("c")

    @pl.when(wid == 0)
    def _():
        pltpu.sync_copy(src_hbm, src_v)
        pltpu.sync_copy(dst_hbm, dst_v)
        s = src_v[...]
        d = dst_v[...]

        deg_v[...] = jnp.zeros((16,), jnp.float32)
        plsc.addupdate_scatter(deg_v, [d], jnp.ones((16,), jnp.float32))

        x = deg_v[...]
        i = lax.bitcast_convert_type(x, jnp.int32)
        i = 0x5F3759DF - (i >> 1)
        y = lax.bitcast_convert_type(i, jnp.float32)
        for _ in range(3):
            y = y * (1.5 - 0.5 * x * y * y)
        deg_v[...] = y

        norm = plsc.load_gather(deg_v, [s]) * plsc.load_gather(deg_v, [d])
        a_v[...] = jnp.zeros((16,), jnp.float32)
        plsc.addupdate_scatter(a_v, [d * 4 + s], norm)
        pltpu.sync_copy(a_v, a_hbm)


def _dense_kernel(a_ref, x_ref, wc1_ref, bc1_ref, wc2_ref, bc2_ref,
                  w1_ref, b1_ref, w2_ref, b2_ref, w3_ref, b3_ref,
                  w4_ref, b4_ref, out_ref):
    A = a_ref[...]
    x = x_ref[...]

    h1 = _dot_exact(A, _dot(x, wc1_ref[...])) + bc1_ref[...]
    h2 = _dot_exact(A, _dot(h1, wc2_ref[...])) + bc2_ref[...]

    x1 = jnp.concatenate([h2[0:1, :], h2[1:2, :], h2[2:3, :], h2[3:4, :]],
                         axis=1)

    x2 = _dot(x1, w1_ref[...]) + b1_ref[...]
    o = _dot(x2, w2_ref[...]) + b2_ref[...]
    o = _dot(o, w3_ref[...]) + b3_ref[...]
    o = o + x1 + x2
    out_ref[...] = jnp.sum(o * w4_ref[...], axis=1, keepdims=True) \
        + b4_ref[...]


@jax.jit
def _run(x, edge_index, Wc1, bc1, Wc2, bc2, W1, b1, W2, b2, W3, b3, W4, b4):
    sl = jnp.arange(4, dtype=jnp.int32)
    src = jnp.concatenate([edge_index[0], sl])
    dst = jnp.concatenate([edge_index[1], sl])
    a_flat = _adjacency_sc(src, dst)
    A = a_flat.reshape(4, 4)

    out = pl.pallas_call(
        _dense_kernel,
        out_shape=jax.ShapeDtypeStruct((1, 1), jnp.float32),
    )(A, x,
      Wc1, bc1.reshape(1, -1),
      Wc2, bc2.reshape(1, -1),
      W1, b1.reshape(1, -1),
      W2, b2.reshape(1, -1),
      W3, b3.reshape(1, -1),
      W4.reshape(1, -1), b4.reshape(1, -1))
    return out.reshape(1)


def kernel(x, edge_index, Wc1, bc1, Wc2, bc2, W1, b1, W2, b2, W3, b3, W4, b4):
    return _run(x, edge_index, Wc1, bc1, Wc2, bc2,
                W1, b1, W2, b2, W3, b3, W4, b4)


# speedup vs baseline: 1.0045x; 1.0045x over previous
import functools

import jax
import jax.numpy as jnp
from jax import lax
from jax.experimental import pallas as pl
from jax.experimental.pallas import tpu as pltpu
from jax.experimental.pallas import tpu_sc as plsc

def _dot(a, b):
    return jnp.dot(a, b, preferred_element_type=jnp.float32,
                   precision=jax.lax.Precision.DEFAULT)


def _dot_exact(a, b):
    return jnp.dot(a, b, preferred_element_type=jnp.float32,
                   precision=jax.lax.Precision.HIGHEST)


_SC_MESH = plsc.VectorSubcoreMesh(core_axis_name="c", subcore_axis_name="s",
                                  num_cores=1, num_subcores=1)


@functools.partial(
    pl.kernel,
    mesh=_SC_MESH,
    compiler_params=pltpu.CompilerParams(needs_layout_passes=False),
    out_type=jax.ShapeDtypeStruct((16,), jnp.float32),
    scratch_types=[
        pltpu.VMEM((16,), jnp.int32),
        pltpu.VMEM((16,), jnp.int32),
        pltpu.VMEM((16,), jnp.float32),
        pltpu.VMEM((16,), jnp.float32),
    ],
)
def _adjacency_sc(src_hbm, dst_hbm, a_hbm, src_v, dst_v, deg_v, a_v):
    wid = lax.axis_index("s") * 2 + lax.axis_index("c")

    @pl.when(wid == 0)
    def _():
        pltpu.sync_copy(src_hbm, src_v)
        pltpu.sync_copy(dst_hbm, dst_v)
        s = src_v[...]
        d = dst_v[...]

        deg_v[...] = jnp.zeros((16,), jnp.float32)
        plsc.addupdate_scatter(deg_v, [d], jnp.ones((16,), jnp.float32))

        x = deg_v[...]
        i = lax.bitcast_convert_type(x, jnp.int32)
        i = 0x5F3759DF - (i >> 1)
        y = lax.bitcast_convert_type(i, jnp.float32)
        for _ in range(3):
            y = y * (1.5 - 0.5 * x * y * y)
        deg_v[...] = y

        norm = plsc.load_gather(deg_v, [s]) * plsc.load_gather(deg_v, [d])
        a_v[...] = jnp.zeros((16,), jnp.float32)
        plsc.addupdate_scatter(a_v, [d * 4 + s], norm)
        pltpu.sync_copy(a_v, a_hbm)


def _dense_kernel(a_ref, x_ref, wc1_ref, bc1_ref, wc2_ref, bc2_ref,
                  w1_ref, b1_ref, w2_ref, b2_ref, w3_ref, b3_ref,
                  w4_ref, b4_ref, out_ref):
    A = a_ref[...]
    x = x_ref[...]

    h1 = _dot_exact(A, _dot(x, wc1_ref[...])) + bc1_ref[...]
    h2 = _dot_exact(A, _dot(h1, wc2_ref[...])) + bc2_ref[...]

    x1 = jnp.concatenate([h2[0:1, :], h2[1:2, :], h2[2:3, :], h2[3:4, :]],
                         axis=1)

    x2 = _dot(x1, w1_ref[...]) + b1_ref[...]
    o = _dot(x2, w2_ref[...]) + b2_ref[...]
    o = _dot(o, w3_ref[...]) + b3_ref[...]
    o = o + x1 + x2
    out_ref[...] = jnp.sum(o * w4_ref[...], axis=1, keepdims=True) \
        + b4_ref[...]


@jax.jit
def _run(x, edge_index, Wc1, bc1, Wc2, bc2, W1, b1, W2, b2, W3, b3, W4, b4):
    sl = jnp.arange(4, dtype=jnp.int32)
    src = jnp.concatenate([edge_index[0], sl])
    dst = jnp.concatenate([edge_index[1], sl])
    a_flat = _adjacency_sc(src, dst)
    A = a_flat.reshape(4, 4)

    out = pl.pallas_call(
        _dense_kernel,
        out_shape=jax.ShapeDtypeStruct((1, 1), jnp.float32),
    )(A, x,
      Wc1, bc1.reshape(1, -1),
      Wc2, bc2.reshape(1, -1),
      W1, b1.reshape(1, -1),
      W2, b2.reshape(1, -1),
      W3, b3.reshape(1, -1),
      W4.reshape(1, -1), b4.reshape(1, -1))
    return out.reshape(1)


def kernel(x, edge_index, Wc1, bc1, Wc2, bc2, W1, b1, W2, b2, W3, b3, W4, b4):
    return _run(x, edge_index, Wc1, bc1, Wc2, bc2,
                W1, b1, W2, b2, W3, b3, W4, b4)
